# baseline (speedup 1.0000x reference)
import jax
import jax.numpy as jnp
from jax.experimental import pallas as pl
from jax.experimental.pallas import tpu as pltpu

B, N, C = 2, 2048, 768
E, H, D, K = 8, 384, 4, 2

_NEG_INF = float("-inf")

_NBG = 8
_BNG = N // _NBG
_NBE = 4
_BNE = N // _NBE


def _gating_kernel(task_ids_ref, x_ref, gw_ref, eps_ref, tkg_ref, tki_ref,
                   s_ref):
    del task_ids_ref
    b = pl.program_id(0)
    nb = pl.program_id(1)
    tw = jnp.dot(x_ref[0], gw_ref[0], preferred_element_type=jnp.float32)
    clean = tw[:, :E]
    raw = tw[:, E:]
    std = jax.nn.softplus(raw) + 0.01
    logits = clean + eps_ref[0] * std
    part = jnp.sum(logits, axis=0, keepdims=True)

    @pl.when(nb == 0)
    def _init():
        s_ref[:, :] = part

    @pl.when(nb != 0)
    def _acc():
        s_ref[:, :] = s_ref[:, :] + part

    @pl.when(nb == _NBG - 1)
    def _finish():
        s = s_ref[:, :]
        iota = jax.lax.broadcasted_iota(jnp.int32, (1, E), 1)
        m2 = jnp.max(s)
        i2 = jnp.min(jnp.where(s == m2, iota, E))
        masked = jnp.where(iota == i2, _NEG_INF, s)
        m1 = jnp.max(masked)
        i1 = jnp.min(jnp.where(masked == m1, iota, E))
        d = m2 - m1
        a = d / (d + 1e-6)
        ena = jnp.exp(-a)
        denom = 1.0 + ena
        tkg_ref[b, 0] = 1.0 / denom
        tkg_ref[b, 1] = ena / denom
        tki_ref[b, 0] = i2
        tki_ref[b, 1] = i1


def _expert_pair(xbf, w1_ref, b1_ref, w2_ref, b2_ref):
    h = jnp.dot(xbf, w1_ref[0].astype(jnp.bfloat16),
                preferred_element_type=jnp.float32) + b1_ref[0]
    h = h * 0.5 * (1.0 + jax.lax.erf(h * 0.7071067811865476))
    return jnp.dot(h.astype(jnp.bfloat16), w2_ref[0].astype(jnp.bfloat16),
                   preferred_element_type=jnp.float32) + b2_ref[0]


def _expert_kernel(tki_ref, x_ref, w1a_ref, b1a_ref, w2a_ref, b2a_ref,
                   w1b_ref, b1b_ref, w2b_ref, b2b_ref, tkg_ref, out_ref):
    del tki_ref
    b = pl.program_id(0)
    xb = x_ref[0]
    xbf = xb.astype(jnp.bfloat16)
    y0 = _expert_pair(xbf, w1a_ref, b1a_ref, w2a_ref, b2a_ref)
    y1 = _expert_pair(xbf, w1b_ref, b1b_ref, w2b_ref, b2b_ref)
    out_ref[0] = xb + tkg_ref[b, 0] * y0 + tkg_ref[b, 1] * y1


@jax.jit
def kernel(x, gate_w, w1, b1, w2, b2, eps, task_ids):
    task_ids = task_ids.astype(jnp.int32)
    b1 = b1.reshape(E, 1, H)
    b2 = b2.reshape(E, 1, C)

    tkg, tki = pl.pallas_call(
        _gating_kernel,
        grid_spec=pltpu.PrefetchScalarGridSpec(
            num_scalar_prefetch=1,
            grid=(B, _NBG),
            in_specs=[
                pl.BlockSpec((1, _BNG, C), lambda b, nb, tids: (b, nb, 0)),
                pl.BlockSpec((1, C, 2 * E), lambda b, nb, tids: (tids[b], 0, 0)),
                pl.BlockSpec((1, _BNG, E), lambda b, nb, tids: (b, nb, 0)),
            ],
            out_specs=[
                pl.BlockSpec(memory_space=pltpu.SMEM),
                pl.BlockSpec(memory_space=pltpu.SMEM),
            ],
            scratch_shapes=[pltpu.VMEM((1, E), jnp.float32)],
        ),
        out_shape=[
            jax.ShapeDtypeStruct((B, K), jnp.float32),
            jax.ShapeDtypeStruct((B, K), jnp.int32),
        ],
        compiler_params=pltpu.CompilerParams(
            dimension_semantics=("arbitrary", "arbitrary"),
        ),
    )(task_ids, x, gate_w, eps)

    out = pl.pallas_call(
        _expert_kernel,
        grid_spec=pltpu.PrefetchScalarGridSpec(
            num_scalar_prefetch=1,
            grid=(B, _NBE),
            in_specs=[
                pl.BlockSpec((1, _BNE, C), lambda b, nb, tki: (b, nb, 0)),
                pl.BlockSpec((1, C, H), lambda b, nb, tki: (tki[b, 0], 0, 0)),
                pl.BlockSpec((1, 1, H), lambda b, nb, tki: (tki[b, 0], 0, 0)),
                pl.BlockSpec((1, H, C), lambda b, nb, tki: (tki[b, 0], 0, 0)),
                pl.BlockSpec((1, 1, C), lambda b, nb, tki: (tki[b, 0], 0, 0)),
                pl.BlockSpec((1, C, H), lambda b, nb, tki: (tki[b, 1], 0, 0)),
                pl.BlockSpec((1, 1, H), lambda b, nb, tki: (tki[b, 1], 0, 0)),
                pl.BlockSpec((1, H, C), lambda b, nb, tki: (tki[b, 1], 0, 0)),
                pl.BlockSpec((1, 1, C), lambda b, nb, tki: (tki[b, 1], 0, 0)),
                pl.BlockSpec(memory_space=pltpu.SMEM),
            ],
            out_specs=pl.BlockSpec((1, _BNE, C), lambda b, nb, tki: (b, nb, 0)),
        ),
        out_shape=jax.ShapeDtypeStruct((B, N, C), jnp.float32),
        compiler_params=pltpu.CompilerParams(
            dimension_semantics=("arbitrary", "arbitrary"),
        ),
    )(tki, x, w1, b1, w2, b2, w1, b1, w2, b2, tkg)
    return out


# speedup vs baseline: 1.0031x; 1.0031x over previous
import jax
import jax.numpy as jnp
from jax.experimental import pallas as pl
from jax.experimental.pallas import tpu as pltpu

B, N, C = 2, 2048, 768
E, H, D, K = 8, 384, 4, 2

_NEG_INF = float("-inf")

_NBG = 8
_BNG = N // _NBG
_NBE = 1
_BNE = N // _NBE


def _gating_kernel(task_ids_ref, x_ref, gw_ref, eps_ref, tkg_ref, tki_ref,
                   s_ref):
    del task_ids_ref
    b = pl.program_id(0)
    nb = pl.program_id(1)
    tw = jnp.dot(x_ref[0], gw_ref[0], preferred_element_type=jnp.float32)
    clean = tw[:, :E]
    raw = tw[:, E:]
    std = jax.nn.softplus(raw) + 0.01
    logits = clean + eps_ref[0] * std
    part = jnp.sum(logits, axis=0, keepdims=True)

    @pl.when(nb == 0)
    def _init():
        s_ref[:, :] = part

    @pl.when(nb != 0)
    def _acc():
        s_ref[:, :] = s_ref[:, :] + part

    @pl.when(nb == _NBG - 1)
    def _finish():
        s = s_ref[:, :]
        iota = jax.lax.broadcasted_iota(jnp.int32, (1, E), 1)
        m2 = jnp.max(s)
        i2 = jnp.min(jnp.where(s == m2, iota, E))
        masked = jnp.where(iota == i2, _NEG_INF, s)
        m1 = jnp.max(masked)
        i1 = jnp.min(jnp.where(masked == m1, iota, E))
        d = m2 - m1
        a = d / (d + 1e-6)
        ena = jnp.exp(-a)
        denom = 1.0 + ena
        tkg_ref[b, 0] = 1.0 / denom
        tkg_ref[b, 1] = ena / denom
        tki_ref[b, 0] = i2
        tki_ref[b, 1] = i1


def _expert_pair(xbf, w1_ref, b1_ref, w2_ref, b2_ref):
    h = jnp.dot(xbf, w1_ref[0].astype(jnp.bfloat16),
                preferred_element_type=jnp.float32) + b1_ref[0]
    h = h * 0.5 * (1.0 + jax.lax.erf(h * 0.7071067811865476))
    return jnp.dot(h.astype(jnp.bfloat16), w2_ref[0].astype(jnp.bfloat16),
                   preferred_element_type=jnp.float32) + b2_ref[0]


def _expert_kernel(tki_ref, x_ref, w1a_ref, b1a_ref, w2a_ref, b2a_ref,
                   w1b_ref, b1b_ref, w2b_ref, b2b_ref, tkg_ref, out_ref):
    del tki_ref
    b = pl.program_id(0)
    xb = x_ref[0]
    xbf = xb.astype(jnp.bfloat16)
    y0 = _expert_pair(xbf, w1a_ref, b1a_ref, w2a_ref, b2a_ref)
    y1 = _expert_pair(xbf, w1b_ref, b1b_ref, w2b_ref, b2b_ref)
    out_ref[0] = xb + tkg_ref[b, 0] * y0 + tkg_ref[b, 1] * y1


@jax.jit
def kernel(x, gate_w, w1, b1, w2, b2, eps, task_ids):
    task_ids = task_ids.astype(jnp.int32)
    b1 = b1.reshape(E, 1, H)
    b2 = b2.reshape(E, 1, C)

    tkg, tki = pl.pallas_call(
        _gating_kernel,
        grid_spec=pltpu.PrefetchScalarGridSpec(
            num_scalar_prefetch=1,
            grid=(B, _NBG),
            in_specs=[
                pl.BlockSpec((1, _BNG, C), lambda b, nb, tids: (b, nb, 0)),
                pl.BlockSpec((1, C, 2 * E), lambda b, nb, tids: (tids[b], 0, 0)),
                pl.BlockSpec((1, _BNG, E), lambda b, nb, tids: (b, nb, 0)),
            ],
            out_specs=[
                pl.BlockSpec(memory_space=pltpu.SMEM),
                pl.BlockSpec(memory_space=pltpu.SMEM),
            ],
            scratch_shapes=[pltpu.VMEM((1, E), jnp.float32)],
        ),
        out_shape=[
            jax.ShapeDtypeStruct((B, K), jnp.float32),
            jax.ShapeDtypeStruct((B, K), jnp.int32),
        ],
        compiler_params=pltpu.CompilerParams(
            dimension_semantics=("arbitrary", "arbitrary"),
        ),
    )(task_ids, x, gate_w, eps)

    out = pl.pallas_call(
        _expert_kernel,
        grid_spec=pltpu.PrefetchScalarGridSpec(
            num_scalar_prefetch=1,
            grid=(B, _NBE),
            in_specs=[
                pl.BlockSpec((1, _BNE, C), lambda b, nb, tki: (b, nb, 0)),
                pl.BlockSpec((1, C, H), lambda b, nb, tki: (tki[b, 0], 0, 0)),
                pl.BlockSpec((1, 1, H), lambda b, nb, tki: (tki[b, 0], 0, 0)),
                pl.BlockSpec((1, H, C), lambda b, nb, tki: (tki[b, 0], 0, 0)),
                pl.BlockSpec((1, 1, C), lambda b, nb, tki: (tki[b, 0], 0, 0)),
                pl.BlockSpec((1, C, H), lambda b, nb, tki: (tki[b, 1], 0, 0)),
                pl.BlockSpec((1, 1, H), lambda b, nb, tki: (tki[b, 1], 0, 0)),
                pl.BlockSpec((1, H, C), lambda b, nb, tki: (tki[b, 1], 0, 0)),
                pl.BlockSpec((1, 1, C), lambda b, nb, tki: (tki[b, 1], 0, 0)),
                pl.BlockSpec(memory_space=pltpu.SMEM),
            ],
            out_specs=pl.BlockSpec((1, _BNE, C), lambda b, nb, tki: (b, nb, 0)),
        ),
        out_shape=jax.ShapeDtypeStruct((B, N, C), jnp.float32),
        compiler_params=pltpu.CompilerParams(
            dimension_semantics=("arbitrary", "arbitrary"),
        ),
    )(tki, x, w1, b1, w2, b2, w1, b1, w2, b2, tkg)
    return out


# speedup vs baseline: 1.2005x; 1.1968x over previous
import jax
import jax.numpy as jnp
from jax.experimental import pallas as pl
from jax.experimental.pallas import tpu as pltpu

B, N, C = 2, 2048, 768
E, H, D, K = 8, 384, 4, 2

_NEG_INF = float("-inf")

_NBG = 1
_BNG = N // _NBG
_NBE = 1
_BNE = N // _NBE


def _gating_kernel(task_ids_ref, x_ref, gw_ref, eps_ref, tkg_ref, tki_ref,
                   s_ref):
    del task_ids_ref
    b = pl.program_id(0)
    nb = pl.program_id(1)
    tw = jnp.dot(x_ref[0], gw_ref[0], preferred_element_type=jnp.float32)
    clean = tw[:, :E]
    raw = tw[:, E:]
    std = jax.nn.softplus(raw) + 0.01
    logits = clean + eps_ref[0] * std
    part = jnp.sum(logits, axis=0, keepdims=True)

    @pl.when(nb == 0)
    def _init():
        s_ref[:, :] = part

    @pl.when(nb != 0)
    def _acc():
        s_ref[:, :] = s_ref[:, :] + part

    @pl.when(nb == _NBG - 1)
    def _finish():
        s = s_ref[:, :]
        iota = jax.lax.broadcasted_iota(jnp.int32, (1, E), 1)
        m2 = jnp.max(s)
        i2 = jnp.min(jnp.where(s == m2, iota, E))
        masked = jnp.where(iota == i2, _NEG_INF, s)
        m1 = jnp.max(masked)
        i1 = jnp.min(jnp.where(masked == m1, iota, E))
        d = m2 - m1
        a = d / (d + 1e-6)
        ena = jnp.exp(-a)
        denom = 1.0 + ena
        tkg_ref[b, 0] = 1.0 / denom
        tkg_ref[b, 1] = ena / denom
        tki_ref[b, 0] = i2
        tki_ref[b, 1] = i1


def _expert_pair(xbf, w1_ref, b1_ref, w2_ref, b2_ref):
    h = jnp.dot(xbf, w1_ref[0].astype(jnp.bfloat16),
                preferred_element_type=jnp.float32) + b1_ref[0]
    h = h * 0.5 * (1.0 + jax.lax.erf(h * 0.7071067811865476))
    return jnp.dot(h.astype(jnp.bfloat16), w2_ref[0].astype(jnp.bfloat16),
                   preferred_element_type=jnp.float32) + b2_ref[0]


def _expert_kernel(tki_ref, x_ref, w1a_ref, b1a_ref, w2a_ref, b2a_ref,
                   w1b_ref, b1b_ref, w2b_ref, b2b_ref, tkg_ref, out_ref):
    del tki_ref
    b = pl.program_id(0)
    xb = x_ref[0]
    xbf = xb.astype(jnp.bfloat16)
    y0 = _expert_pair(xbf, w1a_ref, b1a_ref, w2a_ref, b2a_ref)
    y1 = _expert_pair(xbf, w1b_ref, b1b_ref, w2b_ref, b2b_ref)
    out_ref[0] = xb + tkg_ref[b, 0] * y0 + tkg_ref[b, 1] * y1


@jax.jit
def kernel(x, gate_w, w1, b1, w2, b2, eps, task_ids):
    task_ids = task_ids.astype(jnp.int32)
    b1 = b1.reshape(E, 1, H)
    b2 = b2.reshape(E, 1, C)

    tkg, tki = pl.pallas_call(
        _gating_kernel,
        grid_spec=pltpu.PrefetchScalarGridSpec(
            num_scalar_prefetch=1,
            grid=(B, _NBG),
            in_specs=[
                pl.BlockSpec((1, _BNG, C), lambda b, nb, tids: (b, nb, 0)),
                pl.BlockSpec((1, C, 2 * E), lambda b, nb, tids: (tids[b], 0, 0)),
                pl.BlockSpec((1, _BNG, E), lambda b, nb, tids: (b, nb, 0)),
            ],
            out_specs=[
                pl.BlockSpec(memory_space=pltpu.SMEM),
                pl.BlockSpec(memory_space=pltpu.SMEM),
            ],
            scratch_shapes=[pltpu.VMEM((1, E), jnp.float32)],
        ),
        out_shape=[
            jax.ShapeDtypeStruct((B, K), jnp.float32),
            jax.ShapeDtypeStruct((B, K), jnp.int32),
        ],
        compiler_params=pltpu.CompilerParams(
            dimension_semantics=("arbitrary", "arbitrary"),
        ),
    )(task_ids, x, gate_w, eps)

    out = pl.pallas_call(
        _expert_kernel,
        grid_spec=pltpu.PrefetchScalarGridSpec(
            num_scalar_prefetch=1,
            grid=(B, _NBE),
            in_specs=[
                pl.BlockSpec((1, _BNE, C), lambda b, nb, tki: (b, nb, 0)),
                pl.BlockSpec((1, C, H), lambda b, nb, tki: (tki[b, 0], 0, 0)),
                pl.BlockSpec((1, 1, H), lambda b, nb, tki: (tki[b, 0], 0, 0)),
                pl.BlockSpec((1, H, C), lambda b, nb, tki: (tki[b, 0], 0, 0)),
                pl.BlockSpec((1, 1, C), lambda b, nb, tki: (tki[b, 0], 0, 0)),
                pl.BlockSpec((1, C, H), lambda b, nb, tki: (tki[b, 1], 0, 0)),
                pl.BlockSpec((1, 1, H), lambda b, nb, tki: (tki[b, 1], 0, 0)),
                pl.BlockSpec((1, H, C), lambda b, nb, tki: (tki[b, 1], 0, 0)),
                pl.BlockSpec((1, 1, C), lambda b, nb, tki: (tki[b, 1], 0, 0)),
                pl.BlockSpec(memory_space=pltpu.SMEM),
            ],
            out_specs=pl.BlockSpec((1, _BNE, C), lambda b, nb, tki: (b, nb, 0)),
        ),
        out_shape=jax.ShapeDtypeStruct((B, N, C), jnp.float32),
        compiler_params=pltpu.CompilerParams(
            dimension_semantics=("arbitrary", "arbitrary"),
        ),
    )(tki, x, w1, b1, w2, b2, w1, b1, w2, b2, tkg)
    return out


# speedup vs baseline: 1.2607x; 1.0501x over previous
import jax
import jax.numpy as jnp
from jax.experimental import pallas as pl
from jax.experimental.pallas import tpu as pltpu

B, N, C = 2, 2048, 768
E, H, D, K = 8, 384, 4, 2

_NEG_INF = float("-inf")


def _moe_kernel(task_ids_ref, x_ref, gw_ref, eps_ref, w1_ref, b1_ref,
                w2_ref, b2_ref, out_ref, w1s_ref, w2s_ref,
                sem1a, sem1b, sem2a, sem2b):
    del task_ids_ref
    xb = x_ref[0]

    tw = jnp.dot(xb, gw_ref[0], preferred_element_type=jnp.float32)
    clean = tw[:, :E]
    raw = tw[:, E:]
    std = jax.nn.softplus(raw) + 0.01
    logits = clean + eps_ref[0] * std
    s = jnp.sum(logits, axis=0, keepdims=True)
    iota = jax.lax.broadcasted_iota(jnp.int32, (1, E), 1)
    m2 = jnp.max(s)
    e0 = jnp.min(jnp.where(s == m2, iota, E))
    masked = jnp.where(iota == e0, _NEG_INF, s)
    m1 = jnp.max(masked)
    e1 = jnp.min(jnp.where(masked == m1, iota, E))
    d = m2 - m1
    a = d / (d + 1e-6)
    ena = jnp.exp(-a)
    g0 = 1.0 / (1.0 + ena)
    g1 = ena / (1.0 + ena)

    cp1a = pltpu.make_async_copy(w1_ref.at[e0], w1s_ref.at[0], sem1a)
    cp1b = pltpu.make_async_copy(w1_ref.at[e1], w1s_ref.at[1], sem1b)
    cp2a = pltpu.make_async_copy(w2_ref.at[e0], w2s_ref.at[0], sem2a)
    cp2b = pltpu.make_async_copy(w2_ref.at[e1], w2s_ref.at[1], sem2b)
    cp1a.start()
    cp1b.start()
    cp2a.start()
    cp2b.start()

    xbf = xb.astype(jnp.bfloat16)
    b1a = b1_ref[pl.ds(e0, 1), :]
    b1b = b1_ref[pl.ds(e1, 1), :]
    b2a = b2_ref[pl.ds(e0, 1), :]
    b2b = b2_ref[pl.ds(e1, 1), :]

    cp1a.wait()
    h0 = jnp.dot(xbf, w1s_ref[0].astype(jnp.bfloat16),
                 preferred_element_type=jnp.float32) + b1a
    h0 = h0 * 0.5 * (1.0 + jax.lax.erf(h0 * 0.7071067811865476))
    cp2a.wait()
    y0 = jnp.dot(h0.astype(jnp.bfloat16), w2s_ref[0].astype(jnp.bfloat16),
                 preferred_element_type=jnp.float32) + b2a

    cp1b.wait()
    h1 = jnp.dot(xbf, w1s_ref[1].astype(jnp.bfloat16),
                 preferred_element_type=jnp.float32) + b1b
    h1 = h1 * 0.5 * (1.0 + jax.lax.erf(h1 * 0.7071067811865476))
    cp2b.wait()
    y1 = jnp.dot(h1.astype(jnp.bfloat16), w2s_ref[1].astype(jnp.bfloat16),
                 preferred_element_type=jnp.float32) + b2b

    out_ref[0] = xb + g0 * y0 + g1 * y1


@jax.jit
def kernel(x, gate_w, w1, b1, w2, b2, eps, task_ids):
    task_ids = task_ids.astype(jnp.int32)

    out = pl.pallas_call(
        _moe_kernel,
        grid_spec=pltpu.PrefetchScalarGridSpec(
            num_scalar_prefetch=1,
            grid=(B,),
            in_specs=[
                pl.BlockSpec((1, N, C), lambda b, tids: (b, 0, 0)),
                pl.BlockSpec((1, C, 2 * E), lambda b, tids: (tids[b], 0, 0)),
                pl.BlockSpec((1, N, E), lambda b, tids: (b, 0, 0)),
                pl.BlockSpec(memory_space=pltpu.MemorySpace.HBM),
                pl.BlockSpec((E, H), lambda b, tids: (0, 0)),
                pl.BlockSpec(memory_space=pltpu.MemorySpace.HBM),
                pl.BlockSpec((E, C), lambda b, tids: (0, 0)),
            ],
            out_specs=pl.BlockSpec((1, N, C), lambda b, tids: (b, 0, 0)),
            scratch_shapes=[
                pltpu.VMEM((K, C, H), jnp.float32),
                pltpu.VMEM((K, H, C), jnp.float32),
                pltpu.SemaphoreType.DMA,
                pltpu.SemaphoreType.DMA,
                pltpu.SemaphoreType.DMA,
                pltpu.SemaphoreType.DMA,
            ],
        ),
        out_shape=jax.ShapeDtypeStruct((B, N, C), jnp.float32),
        compiler_params=pltpu.CompilerParams(
            dimension_semantics=("arbitrary",),
        ),
    )(task_ids, x, gate_w, eps, w1, b1, w2, b2)
    return out


# speedup vs baseline: 1.5822x; 1.2550x over previous
import jax
import jax.numpy as jnp
from jax.experimental import pallas as pl
from jax.experimental.pallas import tpu as pltpu

B, N, C = 2, 2048, 768
E, H, D, K = 8, 384, 4, 2

_NEG_INF = float("-inf")
_CH = 512
_NCH = N // _CH


def _gelu(v):
    return v * 0.5 * (1.0 + jax.lax.erf(v * 0.7071067811865476))


def _moe_kernel(tids_ref, x_hbm, gw_ref, epsT_ref, w1_hbm, b1_ref, w2_hbm,
                b2_ref, out_hbm, xv, outv, w1s, w2s, sx, sw, so):
    cps_x = []
    for b in range(B):
        for i in range(_NCH):
            cp = pltpu.make_async_copy(
                x_hbm.at[b, pl.ds(i * _CH, _CH), :],
                xv.at[b, pl.ds(i * _CH, _CH), :],
                sx.at[b * _NCH + i])
            cp.start()
            cps_x.append(cp)

    gate_info = []
    for b in range(B):
        tid = tids_ref[b]
        gwb = gw_ref[pl.ds(tid, 1), :, :][0]
        s = jnp.zeros((1, E), jnp.float32)
        for i in range(_NCH):
            cps_x[b * _NCH + i].wait()
            xc = xv[b, pl.ds(i * _CH, _CH), :]
            twc = jnp.dot(xc, gwb, preferred_element_type=jnp.float32)
            clean = twc[:, :E]
            stdc = jax.nn.softplus(twc[:, E:]) + 0.01
            epc = epsT_ref[b, :, pl.ds(i * _CH, _CH)]
            m = jnp.dot(epc, stdc, preferred_element_type=jnp.float32)
            ir = jax.lax.broadcasted_iota(jnp.int32, (E, E), 0)
            ic = jax.lax.broadcasted_iota(jnp.int32, (E, E), 1)
            diag = jnp.sum(jnp.where(ir == ic, m, 0.0), axis=0, keepdims=True)
            s = s + jnp.sum(clean, axis=0, keepdims=True) + diag
        iota = jax.lax.broadcasted_iota(jnp.int32, (1, E), 1)
        m2 = jnp.max(s)
        e0 = jnp.min(jnp.where(s == m2, iota, E))
        masked = jnp.where(iota == e0, _NEG_INF, s)
        m1 = jnp.max(masked)
        e1 = jnp.min(jnp.where(masked == m1, iota, E))
        d = m2 - m1
        a = d / (d + 1e-6)
        ena = jnp.exp(-a)
        g0 = 1.0 / (1.0 + ena)
        g1 = ena / (1.0 + ena)
        cws = []
        for j, e in enumerate((e0, e1)):
            cp1 = pltpu.make_async_copy(w1_hbm.at[e], w1s.at[b * K + j],
                                        sw.at[b * K + j])
            cp2 = pltpu.make_async_copy(w2_hbm.at[e], w2s.at[b * K + j],
                                        sw.at[B * K + b * K + j])
            cp1.start()
            cp2.start()
            cws.append((cp1, cp2))
        gate_info.append((e0, e1, g0, g1, cws))

    cps_o = []
    for b in range(B):
        e0, e1, g0, g1, cws = gate_info[b]
        cws[0][0].wait()
        cws[0][1].wait()
        cws[1][0].wait()
        cws[1][1].wait()
        w1a = w1s[b * K + 0].astype(jnp.bfloat16)
        w1b = w1s[b * K + 1].astype(jnp.bfloat16)
        w2a = w2s[b * K + 0].astype(jnp.bfloat16)
        w2b = w2s[b * K + 1].astype(jnp.bfloat16)
        b1a = b1_ref[pl.ds(e0, 1), :]
        b1b = b1_ref[pl.ds(e1, 1), :]
        b2a = b2_ref[pl.ds(e0, 1), :]
        b2b = b2_ref[pl.ds(e1, 1), :]
        for i in range(_NCH):
            xc = xv[b, pl.ds(i * _CH, _CH), :]
            xbf = xc.astype(jnp.bfloat16)
            h0 = jnp.dot(xbf, w1a, preferred_element_type=jnp.float32) + b1a
            y0 = jnp.dot(_gelu(h0).astype(jnp.bfloat16), w2a,
                         preferred_element_type=jnp.float32) + b2a
            h1 = jnp.dot(xbf, w1b, preferred_element_type=jnp.float32) + b1b
            y1 = jnp.dot(_gelu(h1).astype(jnp.bfloat16), w2b,
                         preferred_element_type=jnp.float32) + b2b
            outv[b, pl.ds(i * _CH, _CH), :] = xc + g0 * y0 + g1 * y1
            cp = pltpu.make_async_copy(
                outv.at[b, pl.ds(i * _CH, _CH), :],
                out_hbm.at[b, pl.ds(i * _CH, _CH), :],
                so.at[b * _NCH + i])
            cp.start()
            cps_o.append(cp)

    for cp in cps_o:
        cp.wait()


@jax.jit
def kernel(x, gate_w, w1, b1, w2, b2, eps, task_ids):
    task_ids = task_ids.astype(jnp.int32)
    epsT = jnp.transpose(eps, (0, 2, 1))

    out = pl.pallas_call(
        _moe_kernel,
        grid_spec=pltpu.PrefetchScalarGridSpec(
            num_scalar_prefetch=1,
            grid=(1,),
            in_specs=[
                pl.BlockSpec(memory_space=pltpu.MemorySpace.HBM),
                pl.BlockSpec((D, C, 2 * E), lambda i, tids: (0, 0, 0)),
                pl.BlockSpec((B, E, N), lambda i, tids: (0, 0, 0)),
                pl.BlockSpec(memory_space=pltpu.MemorySpace.HBM),
                pl.BlockSpec((E, H), lambda i, tids: (0, 0)),
                pl.BlockSpec(memory_space=pltpu.MemorySpace.HBM),
                pl.BlockSpec((E, C), lambda i, tids: (0, 0)),
            ],
            out_specs=pl.BlockSpec(memory_space=pltpu.MemorySpace.HBM),
            scratch_shapes=[
                pltpu.VMEM((B, N, C), jnp.float32),
                pltpu.VMEM((B, N, C), jnp.float32),
                pltpu.VMEM((B * K, C, H), jnp.float32),
                pltpu.VMEM((B * K, H, C), jnp.float32),
                pltpu.SemaphoreType.DMA((B * _NCH,)),
                pltpu.SemaphoreType.DMA((2 * B * K,)),
                pltpu.SemaphoreType.DMA((B * _NCH,)),
            ],
        ),
        out_shape=jax.ShapeDtypeStruct((B, N, C), jnp.float32),
        compiler_params=pltpu.CompilerParams(
            dimension_semantics=("arbitrary",),
        ),
    )(task_ids, x, gate_w, epsT, w1, b1, w2, b2)
    return out
